# baseline (speedup 1.0000x reference)
import functools

import jax
import jax.numpy as jnp
from jax import lax
from jax.experimental import pallas as pl
from jax.experimental.pallas import tpu as pltpu
from jax.experimental.pallas import tpu_sc as plsc

_NC = 2
_NS = 16
_NW = _NC * _NS
_GB = 128
_NBUF = 8


def _sc_gather(n_chunks, n_rows, d):
    n_outer = n_chunks // _NBUF
    mesh = plsc.VectorSubcoreMesh(core_axis_name="c", subcore_axis_name="s")

    @functools.partial(
        pl.kernel,
        out_type=jax.ShapeDtypeStruct((n_rows, d), jnp.float32),
        mesh=mesh,
        scratch_types=[
            pltpu.VMEM((n_chunks, _GB), jnp.int32),
            pltpu.VMEM((_NBUF, _GB, d), jnp.float32),
            pltpu.SemaphoreType.DMA((_NBUF,)),
            pltpu.SemaphoreType.DMA,
        ],
        compiler_params=pltpu.CompilerParams(use_tc_tiling_on_sc=False),
    )
    def emb(idx_hbm, table_hbm, out_hbm, idx_v, rows_v, gsem, osem):
        wid = lax.axis_index("s") * _NC + lax.axis_index("c")
        pltpu.sync_copy(idx_hbm.at[wid], idx_v)
        base = wid * n_chunks

        def fire(chunk, slot):
            pltpu.async_copy(
                table_hbm.at[idx_v.at[chunk]], rows_v.at[slot], gsem.at[slot]
            )

        def drain(chunk, slot):
            pltpu.make_async_copy(
                table_hbm.at[idx_v.at[chunk]], rows_v.at[slot], gsem.at[slot]
            ).wait()
            pltpu.async_copy(
                rows_v.at[slot], out_hbm.at[pl.ds((base + chunk) * _GB, _GB)], osem
            )
            pltpu.make_async_copy(
                rows_v.at[slot], out_hbm.at[pl.ds((base + chunk) * _GB, _GB)], osem
            ).wait()

        for b in range(_NBUF):
            fire(b, b)

        @pl.loop(0, n_outer - 1)
        def _(i):
            for b in range(_NBUF):
                g = i * _NBUF + b
                drain(g, b)
                fire(g + _NBUF, b)

        for b in range(_NBUF):
            drain((n_outer - 1) * _NBUF + b, b)

    return emb


def kernel(input, table):
    bsz, nf = input.shape
    v, d = table.shape
    tot = bsz * nf
    group = _NW * _GB * _NBUF
    tot_p = ((tot + group - 1) // group) * group
    flat = input.reshape(-1).astype(jnp.int32)
    if tot_p != tot:
        flat = jnp.concatenate([flat, jnp.zeros((tot_p - tot,), jnp.int32)])
    n_chunks = tot_p // (_NW * _GB)
    idx = flat.reshape(_NW, n_chunks, _GB)
    out = _sc_gather(n_chunks, tot_p, d)(idx, table)
    return out[:tot].reshape(bsz, nf, d)
